# Initial kernel scaffold; baseline (speedup 1.0000x reference)
import jax
import jax.numpy as jnp
from jax.experimental import pallas as pl


def kernel(emb_rel, head_idxs, tail_idxs, bins, attn_proj_w, attn_proj_b, attn_bin, attn_vec, aggr_proj_w, aggr_proj_b):
    raise NotImplementedError("write your pallas kernel here")


# speedup vs baseline: 11.2326x; 11.2326x over previous
import functools

import jax
import jax.numpy as jnp
from jax import lax
from jax.experimental import pallas as pl
from jax.experimental.pallas import tpu as pltpu
from jax.experimental.pallas import tpu_sc as plsc

N = 10000
E = 320000
D = 128
H = 8
DH = 16
NB = 10

NC = 2
NS = 16
NW = NC * NS
EW = E // NW
C = 80
NCHUNK = EW // C

HALF_A = 4096
HALF_B = 1024
ABASE = NC * HALF_A
NR_A = 4480
NR_B = 1152

_scp = pltpu.CompilerParams(needs_layout_passes=False, use_tc_tiling_on_sc=False)


def _pre_body(emb_ref, w1t_ref, w2t_ref, wat_ref, b_ref, ab_ref, a_ref, bm_ref):
    x = emb_ref[...]
    a_ref[...] = jnp.dot(x, w1t_ref[...], preferred_element_type=jnp.float32) + b_ref[...]
    bm_ref[:, :D] = jnp.dot(x, w2t_ref[...], preferred_element_type=jnp.float32)
    bm_ref[:, D:] = jnp.dot(x, wat_ref[...], preferred_element_type=jnp.float32) + ab_ref[...]


def _tc_pre(emb, w1t, w2t, wat, b2, ab2):
    rows = 400
    return pl.pallas_call(
        _pre_body,
        grid=(N // rows,),
        in_specs=[
            pl.BlockSpec((rows, D), lambda i: (i, 0)),
            pl.BlockSpec((D, D), lambda i: (0, 0)),
            pl.BlockSpec((D, D), lambda i: (0, 0)),
            pl.BlockSpec((D, D), lambda i: (0, 0)),
            pl.BlockSpec((1, D), lambda i: (0, 0)),
            pl.BlockSpec((1, D), lambda i: (0, 0)),
        ],
        out_specs=[
            pl.BlockSpec((rows, D), lambda i: (i, 0)),
            pl.BlockSpec((rows, 2 * D), lambda i: (i, 0)),
        ],
        out_shape=[
            jax.ShapeDtypeStruct((N, D), jnp.float32),
            jax.ShapeDtypeStruct((N, 2 * D), jnp.float32),
        ],
    )(emb, w1t, w2t, wat, b2, ab2)


def _zero_stripe(zbuf, acc, base, rt):
    zero = jnp.zeros((16,), jnp.float32)

    @pl.loop(0, rt)
    def _zb(i):
        for j in range(D // 16):
            zbuf[i, pl.ds(j * 16, 16)] = zero

    pltpu.sync_copy(zbuf, acc.at[pl.ds(base, rt)])


def _export(acc, num_hbm, s_hbm, c, s, half):
    nrows = half // NS
    nb = s * nrows
    pltpu.sync_copy(acc.at[pl.ds(nb, nrows)], num_hbm.at[c, pl.ds(nb, nrows)])

    @pl.when(s < 8)
    def _sexp():
        srows = half // DH // 8
        sb = s * srows
        pltpu.sync_copy(acc.at[pl.ds(half + sb, srows)],
                        s_hbm.at[c, pl.ds(sb, srows)])


def _route(hidx_v, sidx_v, sidx2_v, base_node, half, dump):
    @pl.loop(0, C // 16)
    def _si(g):
        sl = pl.ds(g * 16, 16)
        lr = hidx_v[sl] - base_node
        valid = jnp.logical_and(lr >= 0, lr < half)
        sidx_v[sl] = jnp.where(valid, lr, dump)
        sidx2_v[sl] = jnp.where(valid, half + lax.shift_right_logical(lr, 4), dump)


def _sca_body(a_hbm, bm_hbm, hidx_hbm, tidx_hbm, bins_hbm, avec_hbm, abin_hbm,
              num_hbm, s_hbm, wmdef_hbm, wdef_hbm, hidx_v, tidx_v, bins_v,
              sidx_v, sidx2_v, a_rows, bm_rows, wm, ws_src, wcomp, avec_v,
              abin_v, zbuf, acc, sem1, sem2):
    c = lax.axis_index("c")
    s = lax.axis_index("s")
    wid = c * NS + s
    base_node = c * HALF_A
    dump = HALF_A + HALF_A // DH

    pltpu.sync_copy(avec_hbm, avec_v)
    pltpu.sync_copy(abin_hbm, abin_v)

    rt = NR_A // NS
    _zero_stripe(zbuf, acc, s * rt, rt)
    plsc.subcore_barrier()

    zero = jnp.zeros((16,), jnp.float32)
    ebase = wid * EW

    @pl.loop(0, NCHUNK)
    def _chunk(ci):
        off = ebase + ci * C
        pltpu.sync_copy(hidx_hbm.at[pl.ds(off, C)], hidx_v)
        pltpu.sync_copy(tidx_hbm.at[pl.ds(off, C)], tidx_v)
        pltpu.sync_copy(bins_hbm.at[pl.ds(off, C)], bins_v)
        cp1 = pltpu.async_copy(a_hbm.at[hidx_v], a_rows, sem1)
        cp2 = pltpu.async_copy(bm_hbm.at[tidx_v], bm_rows, sem2)
        _route(hidx_v, sidx_v, sidx2_v, base_node, HALF_A, dump)
        cp1.wait()
        cp2.wait()

        @pl.loop(0, C)
        def _l1(e):
            for h in range(H):
                sl = pl.ds(h * 16, 16)
                x = a_rows[e, sl] + bm_rows[e, sl]
                y = jnp.maximum(x, x * 0.2)
                a_rows[e, sl] = plsc.cumsum(y * avec_v[sl])
                ws_src[e, sl] = zero

        @pl.loop(0, C // 16)
        def _l2(g):
            eids = g * 16 + lax.iota(jnp.int32, 16)
            sl = pl.ds(g * 16, 16)
            bv = bins_v[sl]
            wcol = jnp.bitwise_and(hidx_v[sl] - base_node, 15) * H
            for h in range(H):
                lastl = jnp.full((16,), h * 16 + 15, jnp.int32)
                raw = plsc.load_gather(a_rows, [eids, lastl])
                bb = plsc.load_gather(abin_v, [bv * H + h])
                w = jnp.exp(raw + bb)
                plsc.store_scatter(wcomp, [eids, jnp.full((16,), h, jnp.int32)], w)
                plsc.store_scatter(ws_src, [eids, wcol + h], w)

        @pl.loop(0, C)
        def _l3(e):
            ev = jnp.full((16,), e, jnp.int32)
            for h in range(H):
                wspl = plsc.load_gather(wcomp, [ev, jnp.full((16,), h, jnp.int32)])
                m = bm_rows[e, pl.ds(D + h * 16, 16)]
                wm[e, pl.ds(h * 16, 16)] = m * wspl

        pltpu.sync_copy(wm, acc.at[sidx_v], add=True)
        pltpu.sync_copy(ws_src, acc.at[sidx2_v], add=True)
        pltpu.sync_copy(wm, wmdef_hbm.at[pl.ds(off, C)])
        pltpu.sync_copy(wcomp, wdef_hbm.at[pl.ds(off, C)])

    plsc.subcore_barrier()
    _export(acc, num_hbm, s_hbm, c, s, HALF_A)


def _sc_phase_a(a_tab, bm_tab, hidx, tidx, bins, avec, abin):
    mesh = plsc.VectorSubcoreMesh(core_axis_name="c", subcore_axis_name="s")
    fn = pl.kernel(
        _sca_body,
        out_type=(
            jax.ShapeDtypeStruct((NC, HALF_A, D), jnp.float32),
            jax.ShapeDtypeStruct((NC, HALF_A // DH, D), jnp.float32),
            jax.ShapeDtypeStruct((E, D), jnp.float32),
            jax.ShapeDtypeStruct((E, DH), jnp.float32),
        ),
        mesh=mesh,
        compiler_params=_scp,
        scratch_types=[
            pltpu.VMEM((C,), jnp.int32),
            pltpu.VMEM((C,), jnp.int32),
            pltpu.VMEM((C,), jnp.int32),
            pltpu.VMEM((C,), jnp.int32),
            pltpu.VMEM((C,), jnp.int32),
            pltpu.VMEM((C, D), jnp.float32),
            pltpu.VMEM((C, 2 * D), jnp.float32),
            pltpu.VMEM((C, D), jnp.float32),
            pltpu.VMEM((C, D), jnp.float32),
            pltpu.VMEM((C, DH), jnp.float32),
            pltpu.VMEM((D,), jnp.float32),
            pltpu.VMEM((NB * H,), jnp.float32),
            pltpu.VMEM((NR_A // NS, D), jnp.float32),
            pltpu.VMEM_SHARED((NR_A, D), jnp.float32),
            pltpu.SemaphoreType.DMA,
            pltpu.SemaphoreType.DMA,
        ],
    )
    return fn(a_tab, bm_tab, hidx, tidx, bins, avec, abin)


def _scb_body(nr, half, cross, wmdef_hbm, wdef_hbm, hidx_hbm, num_hbm, s_hbm,
              hidx_v, sidx_v, sidx2_v, wm_v, wcomp_v, ws_src, zbuf, acc,
              sem1, sem2):
    c = lax.axis_index("c")
    s = lax.axis_index("s")
    dump = half + half // DH
    if cross:
        base_node = c * half
        obase = ((1 - c) * NS + s) * EW
        nchunk = EW // C
    else:
        base_node = ABASE + c * half
        obase = s * (E // NS)
        nchunk = E // NS // C

    rt = nr // NS
    _zero_stripe(zbuf, acc, s * rt, rt)
    plsc.subcore_barrier()

    zero = jnp.zeros((16,), jnp.float32)

    @pl.loop(0, nchunk)
    def _chunk(ci):
        off = obase + ci * C
        pltpu.sync_copy(hidx_hbm.at[pl.ds(off, C)], hidx_v)
        cp1 = pltpu.async_copy(wmdef_hbm.at[pl.ds(off, C)], wm_v, sem1)
        cp2 = pltpu.async_copy(wdef_hbm.at[pl.ds(off, C)], wcomp_v, sem2)
        _route(hidx_v, sidx_v, sidx2_v, base_node, half, dump)

        @pl.loop(0, C)
        def _z(e):
            for j in range(H):
                ws_src[e, pl.ds(j * 16, 16)] = zero

        cp2.wait()

        @pl.loop(0, C // 16)
        def _w(g):
            eids = g * 16 + lax.iota(jnp.int32, 16)
            wcol = jnp.bitwise_and(hidx_v[pl.ds(g * 16, 16)] - base_node, 15) * H
            for h in range(H):
                wv = plsc.load_gather(wcomp_v, [eids, jnp.full((16,), h, jnp.int32)])
                plsc.store_scatter(ws_src, [eids, wcol + h], wv)

        cp1.wait()
        pltpu.sync_copy(wm_v, acc.at[sidx_v], add=True)
        pltpu.sync_copy(ws_src, acc.at[sidx2_v], add=True)

    plsc.subcore_barrier()
    _export(acc, num_hbm, s_hbm, c, s, half)


def _sc_phase_b(wmdef, wdef, hidx, nr, half, cross):
    mesh = plsc.VectorSubcoreMesh(core_axis_name="c", subcore_axis_name="s")
    fn = pl.kernel(
        functools.partial(_scb_body, nr, half, cross),
        out_type=(
            jax.ShapeDtypeStruct((NC, half, D), jnp.float32),
            jax.ShapeDtypeStruct((NC, half // DH, D), jnp.float32),
        ),
        mesh=mesh,
        compiler_params=_scp,
        scratch_types=[
            pltpu.VMEM((C,), jnp.int32),
            pltpu.VMEM((C,), jnp.int32),
            pltpu.VMEM((C,), jnp.int32),
            pltpu.VMEM((C, D), jnp.float32),
            pltpu.VMEM((C, DH), jnp.float32),
            pltpu.VMEM((C, D), jnp.float32),
            pltpu.VMEM((nr // NS, D), jnp.float32),
            pltpu.VMEM_SHARED((nr, D), jnp.float32),
            pltpu.SemaphoreType.DMA,
            pltpu.SemaphoreType.DMA,
        ],
    )
    return fn(wmdef, wdef, hidx)


_RB = 16


def _post_body(na_ref, nb1_ref, nb2_ref, sa_ref, sb1_ref, sb2_ref, out_ref):
    pid = pl.program_id(0)
    in_a = pid < (ABASE // _RB)
    nm = jnp.where(in_a, na_ref[0] + nb1_ref[0], nb2_ref[0])
    s8 = jnp.where(in_a, sa_ref[0] + sb1_ref[0], sb2_ref[0])
    rowi = lax.broadcasted_iota(jnp.int32, (H, D), 0)
    colj = lax.broadcasted_iota(jnp.int32, (H, D), 1)
    erep = jnp.where(colj // DH == rowi, 1.0, 0.0).astype(jnp.float32)
    den = jnp.dot(s8, erep, preferred_element_type=jnp.float32)
    out_ref[...] = nm / (den + 1e-16)


def _tc_post(na, nb1, nb2, sa, sb1, sb2):
    bpc_a = HALF_A // _RB
    bpc_b = HALF_B // _RB

    def amap(i):
        c = jnp.minimum(i // bpc_a, 1)
        return (c, jnp.clip(i - c * bpc_a, 0, bpc_a - 1), 0)

    def bmap(i):
        j = jnp.maximum(i - 2 * bpc_a, 0)
        c = jnp.minimum(j // bpc_b, 1)
        return (c, jnp.clip(j - c * bpc_b, 0, bpc_b - 1), 0)

    return pl.pallas_call(
        _post_body,
        grid=(N // _RB,),
        in_specs=[
            pl.BlockSpec((1, _RB, D), amap),
            pl.BlockSpec((1, _RB, D), amap),
            pl.BlockSpec((1, _RB, D), bmap),
            pl.BlockSpec((1, _RB, H), amap),
            pl.BlockSpec((1, _RB, H), amap),
            pl.BlockSpec((1, _RB, H), bmap),
        ],
        out_specs=pl.BlockSpec((_RB, D), lambda i: (i, 0)),
        out_shape=jax.ShapeDtypeStruct((N, D), jnp.float32),
    )(na, nb1, nb2, sa, sb1, sb2)


@jax.jit
def _stage1(emb_rel, head_idxs, tail_idxs, bins, attn_proj_w, attn_proj_b,
            attn_bin, attn_vec, aggr_proj_w, aggr_proj_b):
    w1t = attn_proj_w[:, :D].T
    w2t = attn_proj_w[:, D:].T
    wat = aggr_proj_w.T
    b2 = attn_proj_b.reshape(1, D)
    ab2 = aggr_proj_b.reshape(1, D)
    a_tab, bm_tab = _tc_pre(emb_rel, w1t, w2t, wat, b2, ab2)
    avec = attn_vec.reshape(-1)
    abin = attn_bin.reshape(-1)
    return _sc_phase_a(a_tab, bm_tab, head_idxs, tail_idxs, bins, avec, abin)


@jax.jit
def _stage2(wmdef, wdef, head_idxs):
    return _sc_phase_b(wmdef, wdef, head_idxs, NR_A, HALF_A, True)


@jax.jit
def _stage2b(wmdef, wdef, head_idxs):
    return _sc_phase_b(wmdef, wdef, head_idxs, NR_B, HALF_B, False)


@jax.jit
def _stage3(num_a, s_a, num_b1, s_b1, num_b2, s_b2):
    sa = s_a.reshape(NC, HALF_A, H)
    sb1 = s_b1.reshape(NC, HALF_A, H)
    sb2 = s_b2.reshape(NC, HALF_B, H)
    return _tc_post(num_a, num_b1, num_b2, sa, sb1, sb2)


def kernel(emb_rel, head_idxs, tail_idxs, bins, attn_proj_w, attn_proj_b,
           attn_bin, attn_vec, aggr_proj_w, aggr_proj_b):
    num_a, s_a, wmdef, wdef = _stage1(emb_rel, head_idxs, tail_idxs, bins,
                                      attn_proj_w, attn_proj_b, attn_bin,
                                      attn_vec, aggr_proj_w, aggr_proj_b)
    num_b1, s_b1 = _stage2(wmdef, wdef, head_idxs)
    num_b2, s_b2 = _stage2b(wmdef, wdef, head_idxs)
    return _stage3(num_a, s_a, num_b1, s_b1, num_b2, s_b2)


# speedup vs baseline: 11.4821x; 1.0222x over previous
import functools

import jax
import jax.numpy as jnp
from jax import lax
from jax.experimental import pallas as pl
from jax.experimental.pallas import tpu as pltpu
from jax.experimental.pallas import tpu_sc as plsc

N = 10000
E = 320000
D = 128
H = 8
DH = 16
NB = 10

NC = 2
NS = 16
NW = NC * NS
EW = E // NW
C = 80
NCHUNK = EW // C

HALF_A = 4096
HALF_B = 1024
ABASE = NC * HALF_A
NR_A = 4480
NR_B = 1152

_scp = pltpu.CompilerParams(needs_layout_passes=False, use_tc_tiling_on_sc=False)


def _pre_body(emb_ref, w1t_ref, w2t_ref, wat_ref, b_ref, ab_ref, a_ref, bm_ref):
    x = emb_ref[...]
    a_ref[...] = jnp.dot(x, w1t_ref[...], preferred_element_type=jnp.float32) + b_ref[...]
    bm_ref[:, :D] = jnp.dot(x, w2t_ref[...], preferred_element_type=jnp.float32)
    bm_ref[:, D:] = jnp.dot(x, wat_ref[...], preferred_element_type=jnp.float32) + ab_ref[...]


def _tc_pre(emb, w1t, w2t, wat, b2, ab2):
    rows = 400
    return pl.pallas_call(
        _pre_body,
        grid=(N // rows,),
        in_specs=[
            pl.BlockSpec((rows, D), lambda i: (i, 0)),
            pl.BlockSpec((D, D), lambda i: (0, 0)),
            pl.BlockSpec((D, D), lambda i: (0, 0)),
            pl.BlockSpec((D, D), lambda i: (0, 0)),
            pl.BlockSpec((1, D), lambda i: (0, 0)),
            pl.BlockSpec((1, D), lambda i: (0, 0)),
        ],
        out_specs=[
            pl.BlockSpec((rows, D), lambda i: (i, 0)),
            pl.BlockSpec((rows, 2 * D), lambda i: (i, 0)),
        ],
        out_shape=[
            jax.ShapeDtypeStruct((N, D), jnp.float32),
            jax.ShapeDtypeStruct((N, 2 * D), jnp.float32),
        ],
    )(emb, w1t, w2t, wat, b2, ab2)


def _zero_stripe(zbuf, acc, base, rt):
    zero = jnp.zeros((16,), jnp.float32)

    @pl.loop(0, rt)
    def _zb(i):
        for j in range(D // 16):
            zbuf[i, pl.ds(j * 16, 16)] = zero

    pltpu.sync_copy(zbuf, acc.at[pl.ds(base, rt)])


def _export(acc, num_hbm, s_hbm, c, s, half):
    nrows = half // NS
    nb = s * nrows
    pltpu.sync_copy(acc.at[pl.ds(nb, nrows)], num_hbm.at[c, pl.ds(nb, nrows)])

    @pl.when(s < 8)
    def _sexp():
        srows = half // DH // 8
        sb = s * srows
        pltpu.sync_copy(acc.at[pl.ds(half + sb, srows)],
                        s_hbm.at[c, pl.ds(sb, srows)])


def _route(hidx_v, sidx_v, sidx2_v, base_node, half, dump):
    @pl.loop(0, C // 16)
    def _si(g):
        sl = pl.ds(g * 16, 16)
        lr = hidx_v[sl] - base_node
        valid = jnp.logical_and(lr >= 0, lr < half)
        sidx_v[sl] = jnp.where(valid, lr, dump)
        sidx2_v[sl] = jnp.where(valid, half + lax.shift_right_logical(lr, 4), dump)


def _sca_body(a_hbm, bm_hbm, hidx_hbm, tidx_hbm, bins_hbm, avec_hbm, abin_hbm,
              num_hbm, s_hbm, wmdef_hbm, wdef_hbm, hidx_v, tidx_v, bins_v,
              sidx_v, sidx2_v, a_rows, bm_rows, wm, ws_src, wcomp, avec_v,
              abin_v, zbuf, acc, sem1, sem2):
    c = lax.axis_index("c")
    s = lax.axis_index("s")
    wid = c * NS + s
    base_node = c * HALF_A
    dump = HALF_A + HALF_A // DH

    pltpu.sync_copy(avec_hbm, avec_v)
    pltpu.sync_copy(abin_hbm, abin_v)

    rt = NR_A // NS
    _zero_stripe(zbuf, acc, s * rt, rt)
    plsc.subcore_barrier()

    zero = jnp.zeros((16,), jnp.float32)
    ebase = wid * EW

    @pl.loop(0, NCHUNK)
    def _chunk(ci):
        off = ebase + ci * C
        ld1 = pltpu.async_copy(hidx_hbm.at[pl.ds(off, C)], hidx_v, sem1)
        ld2 = pltpu.async_copy(tidx_hbm.at[pl.ds(off, C)], tidx_v, sem1)
        ld3 = pltpu.async_copy(bins_hbm.at[pl.ds(off, C)], bins_v, sem1)
        ld1.wait()
        ld2.wait()
        ld3.wait()
        cp1 = pltpu.async_copy(a_hbm.at[hidx_v], a_rows, sem1)
        cp2 = pltpu.async_copy(bm_hbm.at[tidx_v], bm_rows, sem2)
        _route(hidx_v, sidx_v, sidx2_v, base_node, HALF_A, dump)
        cp1.wait()
        cp2.wait()

        @pl.loop(0, C)
        def _l1(e):
            for h in range(H):
                sl = pl.ds(h * 16, 16)
                x = a_rows[e, sl] + bm_rows[e, sl]
                y = jnp.maximum(x, x * 0.2)
                a_rows[e, sl] = plsc.cumsum(y * avec_v[sl])
                ws_src[e, sl] = zero

        @pl.loop(0, C // 16)
        def _l2(g):
            eids = g * 16 + lax.iota(jnp.int32, 16)
            sl = pl.ds(g * 16, 16)
            bv = bins_v[sl]
            wcol = jnp.bitwise_and(hidx_v[sl] - base_node, 15) * H
            for h in range(H):
                lastl = jnp.full((16,), h * 16 + 15, jnp.int32)
                raw = plsc.load_gather(a_rows, [eids, lastl])
                bb = plsc.load_gather(abin_v, [bv * H + h])
                w = jnp.exp(raw + bb)
                plsc.store_scatter(wcomp, [eids, jnp.full((16,), h, jnp.int32)], w)
                plsc.store_scatter(ws_src, [eids, wcol + h], w)

        @pl.loop(0, C)
        def _l3(e):
            ev = jnp.full((16,), e, jnp.int32)
            for h in range(H):
                wspl = plsc.load_gather(wcomp, [ev, jnp.full((16,), h, jnp.int32)])
                m = bm_rows[e, pl.ds(D + h * 16, 16)]
                wm[e, pl.ds(h * 16, 16)] = m * wspl

        st3 = pltpu.async_copy(wm, wmdef_hbm.at[pl.ds(off, C)], sem1)
        st4 = pltpu.async_copy(wcomp, wdef_hbm.at[pl.ds(off, C)], sem2)
        pltpu.sync_copy(wm, acc.at[sidx_v], add=True)
        pltpu.sync_copy(ws_src, acc.at[sidx2_v], add=True)
        st3.wait()
        st4.wait()

    plsc.subcore_barrier()
    _export(acc, num_hbm, s_hbm, c, s, HALF_A)


def _sc_phase_a(a_tab, bm_tab, hidx, tidx, bins, avec, abin):
    mesh = plsc.VectorSubcoreMesh(core_axis_name="c", subcore_axis_name="s")
    fn = pl.kernel(
        _sca_body,
        out_type=(
            jax.ShapeDtypeStruct((NC, HALF_A, D), jnp.float32),
            jax.ShapeDtypeStruct((NC, HALF_A // DH, D), jnp.float32),
            jax.ShapeDtypeStruct((E, D), jnp.float32),
            jax.ShapeDtypeStruct((E, DH), jnp.float32),
        ),
        mesh=mesh,
        compiler_params=_scp,
        scratch_types=[
            pltpu.VMEM((C,), jnp.int32),
            pltpu.VMEM((C,), jnp.int32),
            pltpu.VMEM((C,), jnp.int32),
            pltpu.VMEM((C,), jnp.int32),
            pltpu.VMEM((C,), jnp.int32),
            pltpu.VMEM((C, D), jnp.float32),
            pltpu.VMEM((C, 2 * D), jnp.float32),
            pltpu.VMEM((C, D), jnp.float32),
            pltpu.VMEM((C, D), jnp.float32),
            pltpu.VMEM((C, DH), jnp.float32),
            pltpu.VMEM((D,), jnp.float32),
            pltpu.VMEM((NB * H,), jnp.float32),
            pltpu.VMEM((NR_A // NS, D), jnp.float32),
            pltpu.VMEM_SHARED((NR_A, D), jnp.float32),
            pltpu.SemaphoreType.DMA,
            pltpu.SemaphoreType.DMA,
        ],
    )
    return fn(a_tab, bm_tab, hidx, tidx, bins, avec, abin)


def _scb_body(nr, half, cross, wmdef_hbm, wdef_hbm, hidx_hbm, num_hbm, s_hbm,
              hidx_v, sidx_v, sidx2_v, wm_v, wcomp_v, ws_src, zbuf, acc,
              sem1, sem2, sem3):
    c = lax.axis_index("c")
    s = lax.axis_index("s")
    dump = half + half // DH
    if cross:
        base_node = c * half
        obase = ((1 - c) * NS + s) * EW
        nchunk = EW // C
    else:
        base_node = ABASE + c * half
        obase = s * (E // NS)
        nchunk = E // NS // C

    rt = nr // NS
    _zero_stripe(zbuf, acc, s * rt, rt)
    plsc.subcore_barrier()

    zero = jnp.zeros((16,), jnp.float32)

    @pl.loop(0, nchunk)
    def _chunk(ci):
        off = obase + ci * C
        cp1 = pltpu.async_copy(wmdef_hbm.at[pl.ds(off, C)], wm_v, sem1)
        cp2 = pltpu.async_copy(wdef_hbm.at[pl.ds(off, C)], wcomp_v, sem2)
        ld1 = pltpu.async_copy(hidx_hbm.at[pl.ds(off, C)], hidx_v, sem3)
        ld1.wait()
        _route(hidx_v, sidx_v, sidx2_v, base_node, half, dump)

        @pl.loop(0, C)
        def _z(e):
            for j in range(H):
                ws_src[e, pl.ds(j * 16, 16)] = zero

        cp2.wait()

        @pl.loop(0, C // 16)
        def _w(g):
            eids = g * 16 + lax.iota(jnp.int32, 16)
            wcol = jnp.bitwise_and(hidx_v[pl.ds(g * 16, 16)] - base_node, 15) * H
            for h in range(H):
                wv = plsc.load_gather(wcomp_v, [eids, jnp.full((16,), h, jnp.int32)])
                plsc.store_scatter(ws_src, [eids, wcol + h], wv)

        cp1.wait()
        pltpu.sync_copy(wm_v, acc.at[sidx_v], add=True)
        pltpu.sync_copy(ws_src, acc.at[sidx2_v], add=True)

    plsc.subcore_barrier()
    _export(acc, num_hbm, s_hbm, c, s, half)


def _sc_phase_b(wmdef, wdef, hidx, nr, half, cross):
    mesh = plsc.VectorSubcoreMesh(core_axis_name="c", subcore_axis_name="s")
    fn = pl.kernel(
        functools.partial(_scb_body, nr, half, cross),
        out_type=(
            jax.ShapeDtypeStruct((NC, half, D), jnp.float32),
            jax.ShapeDtypeStruct((NC, half // DH, D), jnp.float32),
        ),
        mesh=mesh,
        compiler_params=_scp,
        scratch_types=[
            pltpu.VMEM((C,), jnp.int32),
            pltpu.VMEM((C,), jnp.int32),
            pltpu.VMEM((C,), jnp.int32),
            pltpu.VMEM((C, D), jnp.float32),
            pltpu.VMEM((C, DH), jnp.float32),
            pltpu.VMEM((C, D), jnp.float32),
            pltpu.VMEM((nr // NS, D), jnp.float32),
            pltpu.VMEM_SHARED((nr, D), jnp.float32),
            pltpu.SemaphoreType.DMA,
            pltpu.SemaphoreType.DMA,
            pltpu.SemaphoreType.DMA,
        ],
    )
    return fn(wmdef, wdef, hidx)


_RB = 16


def _post_body(na_ref, nb1_ref, nb2_ref, sa_ref, sb1_ref, sb2_ref, out_ref):
    pid = pl.program_id(0)
    in_a = pid < (ABASE // _RB)
    nm = jnp.where(in_a, na_ref[0] + nb1_ref[0], nb2_ref[0])
    s8 = jnp.where(in_a, sa_ref[0] + sb1_ref[0], sb2_ref[0])
    rowi = lax.broadcasted_iota(jnp.int32, (H, D), 0)
    colj = lax.broadcasted_iota(jnp.int32, (H, D), 1)
    erep = jnp.where(colj // DH == rowi, 1.0, 0.0).astype(jnp.float32)
    den = jnp.dot(s8, erep, preferred_element_type=jnp.float32)
    out_ref[...] = nm / (den + 1e-16)


def _tc_post(na, nb1, nb2, sa, sb1, sb2):
    bpc_a = HALF_A // _RB
    bpc_b = HALF_B // _RB

    def amap(i):
        c = jnp.minimum(i // bpc_a, 1)
        return (c, jnp.clip(i - c * bpc_a, 0, bpc_a - 1), 0)

    def bmap(i):
        j = jnp.maximum(i - 2 * bpc_a, 0)
        c = jnp.minimum(j // bpc_b, 1)
        return (c, jnp.clip(j - c * bpc_b, 0, bpc_b - 1), 0)

    return pl.pallas_call(
        _post_body,
        grid=(N // _RB,),
        in_specs=[
            pl.BlockSpec((1, _RB, D), amap),
            pl.BlockSpec((1, _RB, D), amap),
            pl.BlockSpec((1, _RB, D), bmap),
            pl.BlockSpec((1, _RB, H), amap),
            pl.BlockSpec((1, _RB, H), amap),
            pl.BlockSpec((1, _RB, H), bmap),
        ],
        out_specs=pl.BlockSpec((_RB, D), lambda i: (i, 0)),
        out_shape=jax.ShapeDtypeStruct((N, D), jnp.float32),
    )(na, nb1, nb2, sa, sb1, sb2)


@jax.jit
def _stage1(emb_rel, head_idxs, tail_idxs, bins, attn_proj_w, attn_proj_b,
            attn_bin, attn_vec, aggr_proj_w, aggr_proj_b):
    w1t = attn_proj_w[:, :D].T
    w2t = attn_proj_w[:, D:].T
    wat = aggr_proj_w.T
    b2 = attn_proj_b.reshape(1, D)
    ab2 = aggr_proj_b.reshape(1, D)
    a_tab, bm_tab = _tc_pre(emb_rel, w1t, w2t, wat, b2, ab2)
    avec = attn_vec.reshape(-1)
    abin = attn_bin.reshape(-1)
    return _sc_phase_a(a_tab, bm_tab, head_idxs, tail_idxs, bins, avec, abin)


@jax.jit
def _stage2(wmdef, wdef, head_idxs):
    return _sc_phase_b(wmdef, wdef, head_idxs, NR_A, HALF_A, True)


@jax.jit
def _stage2b(wmdef, wdef, head_idxs):
    return _sc_phase_b(wmdef, wdef, head_idxs, NR_B, HALF_B, False)


@jax.jit
def _stage3(num_a, s_a, num_b1, s_b1, num_b2, s_b2):
    sa = s_a.reshape(NC, HALF_A, H)
    sb1 = s_b1.reshape(NC, HALF_A, H)
    sb2 = s_b2.reshape(NC, HALF_B, H)
    return _tc_post(num_a, num_b1, num_b2, sa, sb1, sb2)


def kernel(emb_rel, head_idxs, tail_idxs, bins, attn_proj_w, attn_proj_b,
           attn_bin, attn_vec, aggr_proj_w, aggr_proj_b):
    num_a, s_a, wmdef, wdef = _stage1(emb_rel, head_idxs, tail_idxs, bins,
                                      attn_proj_w, attn_proj_b, attn_bin,
                                      attn_vec, aggr_proj_w, aggr_proj_b)
    num_b1, s_b1 = _stage2(wmdef, wdef, head_idxs)
    num_b2, s_b2 = _stage2b(wmdef, wdef, head_idxs)
    return _stage3(num_a, s_a, num_b1, s_b1, num_b2, s_b2)
